# baseline (speedup 1.0000x reference)
import jax
import jax.numpy as jnp
from jax import lax
from jax.experimental import pallas as pl
from jax.experimental.pallas import tpu as pltpu

B, N = 16, 4096
X_DIM, Y_DIM, H_DIM, R_DIM = 16, 16, 64, 64
PACK = 128 // X_DIM
PROWS = N // PACK
NPAIR = PACK // 2
KC = 2
CR = PROWS // KC


def _bd_mask(rows, cols, rblk, cblk):
    ri = lax.broadcasted_iota(jnp.int32, (rows, cols), 0) // rblk
    ci = lax.broadcasted_iota(jnp.int32, (rows, cols), 1) // cblk
    return (ri == ci).astype(jnp.float32)


def _body(x_ref, y_ref, m_ref, w1_ref, b1_ref, w2_ref, b2_ref, w3_ref,
          b3_ref, out_ref, acc_ref, cnt_ref):
    w1 = w1_ref[...]
    w1x_bd = (jnp.tile(w1[:X_DIM], (PACK, PACK))
              * _bd_mask(128, 512, 16, 64)).astype(jnp.bfloat16)
    w1y_bd = (jnp.tile(w1[X_DIM:], (PACK, PACK))
              * _bd_mask(128, 512, 16, 64)).astype(jnp.bfloat16)
    w2_bd = (jnp.tile(w2_ref[...], (2, 2))
             * _bd_mask(128, 128, 64, 64)).astype(jnp.bfloat16)
    b1t = jnp.tile(b1_ref[...], (1, PACK))
    b2t = jnp.tile(b2_ref[...], (1, 2))
    e_mat = _bd_mask(PACK, PACK * H_DIM, 1, H_DIM).astype(jnp.bfloat16)

    k = pl.program_id(1)
    px = x_ref[0]
    py = y_ref[0]
    mp = m_ref[0]

    h = jnp.dot(px, w1x_bd, preferred_element_type=jnp.float32)
    h = h + jnp.dot(py, w1y_bd, preferred_element_type=jnp.float32)
    h = jnp.maximum(h + b1t, 0.0)
    mexp = jnp.dot(mp, e_mat, preferred_element_type=jnp.float32)
    acc = jnp.zeros((1, 2 * H_DIM), dtype=jnp.float32)
    for p in range(NPAIR):
        g = h[:, 2 * H_DIM * p:2 * H_DIM * (p + 1)].astype(jnp.bfloat16)
        h2 = jnp.dot(g, w2_bd, preferred_element_type=jnp.float32)
        h2 = jnp.maximum(h2 + b2t, 0.0)
        mm = mexp[:, 2 * H_DIM * p:2 * H_DIM * (p + 1)]
        acc = acc + jnp.sum(h2 * mm, axis=0, keepdims=True)
    cnt = jnp.sum(mp.astype(jnp.float32))

    @pl.when(k == 0)
    def _init():
        acc_ref[...] = acc
        cnt_ref[...] = cnt.reshape(1, 1)

    @pl.when(k > 0)
    def _accum():
        acc_ref[...] += acc
        cnt_ref[...] += cnt.reshape(1, 1)

    @pl.when(k == KC - 1)
    def _fin():
        a = acc_ref[...]
        s = a[:, :H_DIM] + a[:, H_DIM:]
        c = cnt_ref[0, 0]
        r = jnp.dot(s, w3_ref[...], preferred_element_type=jnp.float32)
        r = r + c * b3_ref[...]
        out_ref[0] = r / jnp.maximum(c, 1.0)


def kernel(x, y, mask, W1, b1, W2, b2, W3, b3):
    xd = x.astype(jnp.bfloat16).reshape(B, PROWS, 128)
    yd = y.astype(jnp.bfloat16).reshape(B, PROWS, 128)
    mp = mask.astype(jnp.bfloat16).reshape(B, PROWS, PACK)
    b1r = b1.reshape(1, H_DIM)
    b2r = b2.reshape(1, H_DIM)
    b3r = b3.reshape(1, R_DIM)

    out = pl.pallas_call(
        _body,
        grid=(B, KC),
        in_specs=[
            pl.BlockSpec((1, CR, 128), lambda b, k: (b, k, 0)),
            pl.BlockSpec((1, CR, 128), lambda b, k: (b, k, 0)),
            pl.BlockSpec((1, CR, PACK), lambda b, k: (b, k, 0)),
            pl.BlockSpec((X_DIM + Y_DIM, H_DIM), lambda b, k: (0, 0)),
            pl.BlockSpec((1, H_DIM), lambda b, k: (0, 0)),
            pl.BlockSpec((H_DIM, H_DIM), lambda b, k: (0, 0)),
            pl.BlockSpec((1, H_DIM), lambda b, k: (0, 0)),
            pl.BlockSpec((H_DIM, R_DIM), lambda b, k: (0, 0)),
            pl.BlockSpec((1, R_DIM), lambda b, k: (0, 0)),
        ],
        out_specs=pl.BlockSpec((1, 1, R_DIM), lambda b, k: (b, 0, 0)),
        out_shape=jax.ShapeDtypeStruct((B, 1, R_DIM), jnp.float32),
        scratch_shapes=[
            pltpu.VMEM((1, 2 * H_DIM), jnp.float32),
            pltpu.VMEM((1, 1), jnp.float32),
        ],
        compiler_params=pltpu.CompilerParams(
            dimension_semantics=("arbitrary", "arbitrary"),
        ),
    )(xd, yd, mp, W1, b1r, W2, b2r, W3, b3r)
    return out.reshape(B, R_DIM)
